# baseline (speedup 1.0000x reference)
import jax
import jax.numpy as jnp
from jax.experimental import pallas as pl
from jax.experimental.pallas import tpu as pltpu

_BATCH = 1024
_REP = 8
_NSEM = 8


def _dma_bcast_body(rep_hbm, out_ref, *sems):
    n = _BATCH // _REP
    copies = [
        pltpu.make_async_copy(
            rep_hbm, out_ref.at[pl.ds(k * _REP, _REP)], sems[k % _NSEM]
        )
        for k in range(n)
    ]
    for c in copies:
        c.start()
    for c in copies:
        c.wait()


def kernel(embedding, batch_size):
    del batch_size
    v, d = embedding.shape
    flat = v * d

    rep_block = jnp.broadcast_to(embedding.reshape(1, flat), (_REP, flat))
    out = pl.pallas_call(
        _dma_bcast_body,
        in_specs=[pl.BlockSpec(memory_space=pl.ANY)],
        out_specs=pl.BlockSpec(memory_space=pl.ANY),
        out_shape=jax.ShapeDtypeStruct((_BATCH, flat), jnp.float32),
        scratch_shapes=[pltpu.SemaphoreType.DMA] * _NSEM,
    )(rep_block)
    return out.reshape(_BATCH, v, d)


# speedup vs baseline: 25.3410x; 25.3410x over previous
import jax
import jax.numpy as jnp
from jax.experimental import pallas as pl
from jax.experimental.pallas import tpu as pltpu

_BATCH = 1024
_BT = 64


def _bcast_body(emb_ref, out_ref):
    out_ref[...] = jnp.broadcast_to(emb_ref[...][None, :], out_ref.shape)


def kernel(embedding, batch_size):
    del batch_size
    v, d = embedding.shape
    flat = v * d

    out = pl.pallas_call(
        _bcast_body,
        grid=(_BATCH // _BT,),
        in_specs=[pl.BlockSpec((flat,), lambda i: (0,))],
        out_specs=pl.BlockSpec((_BT, flat), lambda i: (i, 0)),
        out_shape=jax.ShapeDtypeStruct((_BATCH, flat), jnp.float32),
        compiler_params=pltpu.CompilerParams(
            dimension_semantics=("arbitrary",),
        ),
    )(embedding.reshape(flat))
    return out.reshape(_BATCH, v, d)


# speedup vs baseline: 25.5376x; 1.0078x over previous
import jax
import jax.numpy as jnp
from jax.experimental import pallas as pl
from jax.experimental.pallas import tpu as pltpu

_BATCH = 1024
_BT = 128


def _bcast_body(emb_ref, out_ref):
    out_ref[...] = jnp.broadcast_to(emb_ref[...][None, :], out_ref.shape)


def kernel(embedding, batch_size):
    del batch_size
    v, d = embedding.shape
    flat = v * d

    out = pl.pallas_call(
        _bcast_body,
        grid=(_BATCH // _BT,),
        in_specs=[pl.BlockSpec((flat,), lambda i: (0,))],
        out_specs=pl.BlockSpec((_BT, flat), lambda i: (i, 0)),
        out_shape=jax.ShapeDtypeStruct((_BATCH, flat), jnp.float32),
        compiler_params=pltpu.CompilerParams(
            dimension_semantics=("arbitrary",),
        ),
    )(embedding.reshape(flat))
    return out.reshape(_BATCH, v, d)
